# Initial kernel scaffold; baseline (speedup 1.0000x reference)
import functools

import jax
import jax.numpy as jnp
from jax import lax
from jax.experimental import pallas as pl
from jax.experimental.pallas import tpu as pltpu
from jax.experimental.pallas import tpu_sc as plsc

N = 10000
E = 320000
D = 128

NC = 2
NS = 16
NW = NC * NS
EPW = E // NW
C = 80
NCH = EPW // C
RPT = N // NS
ZR = 125

_mesh = plsc.VectorSubcoreMesh(core_axis_name="c", subcore_axis_name="s")


@functools.partial(
    pl.kernel,
    out_type=jax.ShapeDtypeStruct((NW, 2, N), jnp.float32),
    mesh=_mesh,
    scratch_types=[
        pltpu.VMEM((2, EPW), jnp.int32),
        pltpu.VMEM((2, N), jnp.float32),
    ],
)
def _deg_kernel(edges_hbm, out_hbm, idx_v, cnt_v):
    cid = lax.axis_index("c")
    sid = lax.axis_index("s")
    wid = sid * NC + cid
    pltpu.sync_copy(edges_hbm.at[0, wid], idx_v.at[0])
    pltpu.sync_copy(edges_hbm.at[1, wid], idx_v.at[1])

    zeros = jnp.zeros((16,), jnp.float32)

    def zbody(i, carry):
        cnt_v[0, pl.ds(i * 16, 16)] = zeros
        cnt_v[1, pl.ds(i * 16, 16)] = zeros
        return carry

    lax.fori_loop(0, N // 16, zbody, 0)

    ones = jnp.ones((16,), jnp.float32)

    def body(i, carry):
        s = idx_v[0, pl.ds(i * 16, 16)]
        d = idx_v[1, pl.ds(i * 16, 16)]
        plsc.addupdate_scatter(cnt_v.at[0], [s], ones)
        plsc.addupdate_scatter(cnt_v.at[1], [d], ones)
        return carry

    lax.fori_loop(0, EPW // 16, body, 0)
    pltpu.sync_copy(cnt_v.at[0], out_hbm.at[wid, 0])
    pltpu.sync_copy(cnt_v.at[1], out_hbm.at[wid, 1])


@functools.partial(
    pl.kernel,
    out_type=jax.ShapeDtypeStruct((NC, N, D), jnp.float32),
    mesh=_mesh,
    scratch_types=[
        pltpu.VMEM((NCH, C), jnp.int32),
        pltpu.VMEM((NCH, C), jnp.int32),
        pltpu.VMEM((C, D), jnp.float32),
        pltpu.VMEM((ZR, D), jnp.float32),
        pltpu.VMEM_SHARED((N, D), jnp.float32),
        pltpu.SemaphoreType.DMA,
    ],
)
def _agg_kernel(u_hbm, src_hbm, dst_hbm, out_hbm, sidx, didx, rows, zbuf, acc, sem):
    cid = lax.axis_index("c")
    sid = lax.axis_index("s")
    wid = sid * NC + cid
    pltpu.sync_copy(src_hbm.at[wid], sidx)
    pltpu.sync_copy(dst_hbm.at[wid], didx)

    zeros = jnp.zeros((16,), jnp.float32)

    def zb(i, carry):
        zbuf[i // 8, pl.ds((i % 8) * 16, 16)] = zeros
        return carry

    lax.fori_loop(0, ZR * (D // 16), zb, 0)

    def zs(i, carry):
        pltpu.sync_copy(zbuf, acc.at[pl.ds(sid * RPT + i * ZR, ZR)])
        return carry

    lax.fori_loop(0, RPT // ZR, zs, 0)
    plsc.subcore_barrier()

    def body(i, carry):
        pltpu.async_copy(u_hbm.at[sidx.at[i]], rows, sem).wait()
        pltpu.sync_copy(rows, acc.at[didx.at[i]], add=True)
        return carry

    lax.fori_loop(0, NCH, body, 0)
    plsc.subcore_barrier()
    pltpu.sync_copy(
        acc.at[pl.ds(sid * RPT, RPT)], out_hbm.at[cid, pl.ds(sid * RPT, RPT)]
    )


def _a0_body(cnt_ref, rr_ref):
    deg = jnp.sum(cnt_ref[...], axis=0)
    rr_ref[...] = lax.rsqrt(jnp.maximum(deg, 1.0))


def _a1_body(x_ref, routc_ref, w1_ref, u_ref):
    u_ref[...] = jnp.dot(
        x_ref[...] * routc_ref[...], w1_ref[...], preferred_element_type=jnp.float32
    )


def _b_body(p_ref, rinc_ref, routc_ref, b1_ref, w2_ref, v_ref):
    agg = (p_ref[0] + p_ref[1]) * rinc_ref[...]
    y1 = jnp.tanh(agg + b1_ref[...])
    v_ref[...] = jnp.dot(
        y1 * routc_ref[...], w2_ref[...], preferred_element_type=jnp.float32
    )


def _c_body(p_ref, rinc_ref, b2_ref, o_ref):
    t = (p_ref[0] + p_ref[1]) * rinc_ref[...] + b2_ref[...]
    o_ref[...] = jnp.concatenate([t[:, D // 2:], -t[:, : D // 2]], axis=1)


_a0_call = pl.pallas_call(
    _a0_body, out_shape=jax.ShapeDtypeStruct((2, N), jnp.float32))
_a1_call = pl.pallas_call(
    _a1_body, out_shape=jax.ShapeDtypeStruct((N, D), jnp.float32))
_b_call = pl.pallas_call(
    _b_body, out_shape=jax.ShapeDtypeStruct((N, D), jnp.float32))
_c_call = pl.pallas_call(
    _c_body, out_shape=jax.ShapeDtypeStruct((N, D), jnp.float32))


def kernel(x, edge_index, W1, b1, W2, b2):
    edges2 = edge_index.reshape(2, NW, EPW)
    src3 = edge_index[0].reshape(NW, NCH, C)
    dst3 = edge_index[1].reshape(NW, NCH, C)

    cnts = _deg_kernel(edges2)
    rr = _a0_call(cnts)
    routc = rr[0].reshape(N, 1)
    rinc = rr[1].reshape(N, 1)

    u = _a1_call(x, routc, W1)
    p1 = _agg_kernel(u, src3, dst3)
    v = _b_call(p1, rinc, routc, b1.reshape(1, D), W2)
    p2 = _agg_kernel(v, src3, dst3)
    out = _c_call(p2, rinc, b2.reshape(1, D))
    return out


# speedup vs baseline: 4.0963x; 4.0963x over previous
import functools

import jax
import jax.numpy as jnp
from jax import lax
from jax.experimental import pallas as pl
from jax.experimental.pallas import tpu as pltpu
from jax.experimental.pallas import tpu_sc as plsc

N = 10000
E = 320000
D = 128

NC = 2
NS = 16
NW = NC * NS
EPW = E // NW
C = 128
EPS = E // NS
NCH = 157
PAD = NCH * C - EPS
NP = 10240
HALF = NP // NC
ACC = 5376
GARBAGE = HALF
ZPT = ACC // NS
ZR = ZPT // 2
CPT = HALF // NS

_mesh = plsc.VectorSubcoreMesh(
    core_axis_name="c", subcore_axis_name="s", num_cores=NC, num_subcores=NS)
_sc_params = pltpu.CompilerParams(needs_layout_passes=False)


@functools.partial(
    pl.kernel,
    out_type=jax.ShapeDtypeStruct((NW, 2, N), jnp.float32),
    mesh=_mesh,
    compiler_params=_sc_params,
    scratch_types=[
        pltpu.VMEM((2, EPW), jnp.int32),
        pltpu.VMEM((N,), jnp.float32),
        pltpu.VMEM((N,), jnp.float32),
    ],
)
def _deg_kernel(edges_hbm, out_hbm, idx_v, cnt_s, cnt_d):
    cid = lax.axis_index("c")
    sid = lax.axis_index("s")
    wid = sid * NC + cid
    pltpu.sync_copy(edges_hbm.at[0, wid], idx_v.at[0])
    pltpu.sync_copy(edges_hbm.at[1, wid], idx_v.at[1])

    zeros = jnp.zeros((16,), jnp.float32)

    def zbody(i, carry):
        cnt_s[pl.ds(i * 16, 16)] = zeros
        cnt_d[pl.ds(i * 16, 16)] = zeros
        return carry

    lax.fori_loop(0, N // 16, zbody, 0)

    ones = jnp.ones((16,), jnp.float32)

    def body(i, carry):
        s = idx_v[0, pl.ds(i * 16, 16)]
        d = idx_v[1, pl.ds(i * 16, 16)]
        plsc.addupdate_scatter(cnt_s, [s], ones)
        plsc.addupdate_scatter(cnt_d, [d], ones)
        return carry

    lax.fori_loop(0, EPW // 16, body, 0)
    pltpu.sync_copy(cnt_s, out_hbm.at[wid, 0])
    pltpu.sync_copy(cnt_d, out_hbm.at[wid, 1])


@functools.partial(
    pl.kernel,
    out_type=jax.ShapeDtypeStruct((NP, D), jnp.float32),
    mesh=_mesh,
    compiler_params=_sc_params,
    scratch_types=[
        pltpu.VMEM((NCH, C), jnp.int32),
        pltpu.VMEM((NCH, C), jnp.int32),
        pltpu.VMEM((C, D), jnp.float32),
        pltpu.VMEM((ZR, D), jnp.float32),
        pltpu.VMEM_SHARED((ACC, D), jnp.float32),
        pltpu.SemaphoreType.DMA,
    ],
)
def _agg_kernel(u_hbm, src_hbm, dst_hbm, out_hbm, sidx, didx, rows, zbuf, acc, sem):
    cid = lax.axis_index("c")
    sid = lax.axis_index("s")
    pltpu.sync_copy(src_hbm.at[sid], sidx)
    pltpu.sync_copy(dst_hbm.at[sid], didx)

    zeros = jnp.zeros((16,), jnp.float32)

    def zb(i, carry):
        zbuf[i // 8, pl.ds((i % 8) * 16, 16)] = zeros
        return carry

    lax.fori_loop(0, ZR * (D // 16), zb, 0)

    def zs(i, carry):
        pltpu.sync_copy(zbuf, acc.at[pl.ds(sid * ZPT + i * ZR, ZR)])
        return carry

    lax.fori_loop(0, ZPT // ZR, zs, 0)

    base = cid * HALF
    VPC = C // 16

    def tr(k, carry):
        i = k // VPC
        j = k % VPC
        d = didx[i, pl.ds(j * 16, 16)] - base
        ok = (d >= 0) & (d < HALF)
        didx[i, pl.ds(j * 16, 16)] = jnp.where(ok, d, GARBAGE)
        return carry

    lax.fori_loop(0, NCH * VPC, tr, 0)
    plsc.subcore_barrier()

    def body(i, carry):
        pltpu.async_copy(u_hbm.at[sidx.at[i]], rows, sem).wait()
        pltpu.sync_copy(rows, acc.at[didx.at[i]], add=True)
        return carry

    lax.fori_loop(0, NCH, body, 0)
    plsc.subcore_barrier()
    pltpu.sync_copy(
        acc.at[pl.ds(sid * CPT, CPT)],
        out_hbm.at[pl.ds(cid * HALF + sid * CPT, CPT)],
    )


def _a0_body(cnt_ref, rr_ref):
    deg = jnp.sum(cnt_ref[...], axis=0)
    rr_ref[...] = lax.rsqrt(jnp.maximum(deg, 1.0))


def _a1_body(x_ref, routc_ref, w1_ref, u_ref):
    u_ref[...] = jnp.dot(
        x_ref[...] * routc_ref[...], w1_ref[...], preferred_element_type=jnp.float32
    )


def _b_body(p_ref, rinc_ref, routc_ref, b1_ref, w2_ref, v_ref):
    agg = p_ref[...] * rinc_ref[...]
    y1 = jnp.tanh(agg + b1_ref[...])
    v_ref[...] = jnp.dot(
        y1 * routc_ref[...], w2_ref[...], preferred_element_type=jnp.float32
    )


def _c_body(p_ref, rinc_ref, b2_ref, o_ref):
    t = p_ref[...] * rinc_ref[...] + b2_ref[...]
    o_ref[...] = jnp.concatenate([t[:, D // 2:], -t[:, : D // 2]], axis=1)


_a0_call = pl.pallas_call(
    _a0_body, out_shape=jax.ShapeDtypeStruct((2, N), jnp.float32))
_a1_call = pl.pallas_call(
    _a1_body, out_shape=jax.ShapeDtypeStruct((N, D), jnp.float32))
_b_call = pl.pallas_call(
    _b_body, out_shape=jax.ShapeDtypeStruct((N, D), jnp.float32))
_c_call = pl.pallas_call(
    _c_body, out_shape=jax.ShapeDtypeStruct((N, D), jnp.float32))


def kernel(x, edge_index, W1, b1, W2, b2):
    edges2 = edge_index.reshape(2, NW, EPW)

    cnts = _deg_kernel(edges2)
    rr = _a0_call(cnts)
    routc = rr[0].reshape(N, 1)
    rinc = rr[1].reshape(N, 1)

    src_pad = jnp.concatenate(
        [edge_index[0].reshape(NS, EPS),
         jnp.zeros((NS, PAD), jnp.int32)], axis=1)
    dst_pad = jnp.concatenate(
        [edge_index[1].reshape(NS, EPS),
         jnp.full((NS, PAD), N, jnp.int32)], axis=1)
    src3 = src_pad.reshape(NS, NCH, C)
    dst3 = dst_pad.reshape(NS, NCH, C)

    u = _a1_call(x, routc, W1)
    p1 = _agg_kernel(u, src3, dst3)[:N]
    v = _b_call(p1, rinc, routc, b1.reshape(1, D), W2)
    p2 = _agg_kernel(v, src3, dst3)[:N]
    out = _c_call(p2, rinc, b2.reshape(1, D))
    return out
